# baseline (speedup 1.0000x reference)
import functools

import jax
import jax.numpy as jnp
from jax import lax
from jax.experimental import pallas as pl
from jax.experimental.pallas import tpu as pltpu
from jax.experimental.pallas import tpu_sc as plsc

N = 10000
NP = 10112
E = 320000
CH = 128
NCHUNK = 2560
EP = NCHUNK * CH
KB = 16
NBLK = NCHUNK // KB
NSUB = 16
NCORE = 2
RPT = NP // NSUB
BN = 632
GRID = NP // BN


def _seg_sum(table, idx_src, idx_dst, width, es):
    nblk = NBLK // (NCORE * NSUB) if es else NBLK // NSUB
    out_type = jax.ShapeDtypeStruct((NCORE * NP, width), jnp.float32)
    NBUF = 2
    scratch = [
        pltpu.VMEM((KB, CH), jnp.int32),
        pltpu.VMEM((KB, CH), jnp.int32),
    ] + [pltpu.VMEM((CH, width), jnp.float32) for _ in range(NBUF)] + [
        pltpu.VMEM_SHARED((NP, width), jnp.float32),
    ] + [pltpu.SemaphoreType.DMA for _ in range(NBUF)]

    mesh = plsc.VectorSubcoreMesh(core_axis_name="c", subcore_axis_name="s",
                                  num_cores=NCORE, num_subcores=NSUB)

    def body(table_r, src_r, dst_r, zw_r, out_r, src_v, dst_v, *rest):
        rows = rest[:NBUF]
        acc = rest[NBUF]
        sems = rest[NBUF + 1:]
        c = lax.axis_index("c")
        s = lax.axis_index("s")
        pltpu.sync_copy(zw_r, acc.at[pl.ds(s * RPT, RPT)])
        if es:
            src_base = (s * NCORE + c) * nblk * KB
            dst_base = src_base
        else:
            src_base = c * NCHUNK + s * nblk * KB
            dst_base = s * nblk * KB
        plsc.subcore_barrier()

        @pl.loop(0, nblk)
        def _blk(b):
            srow = pl.multiple_of(src_base + b * KB, 8)
            drow = pl.multiple_of(dst_base + b * KB, 8)
            pltpu.sync_copy(src_r.at[pl.ds(srow, KB)], src_v)
            pltpu.sync_copy(dst_r.at[pl.ds(drow, KB)], dst_v)

            @pl.loop(0, KB // NBUF)
            def _grp(g):
                cps = [pltpu.async_copy(table_r.at[src_v.at[g * NBUF + k]],
                                        rows[k], sems[k])
                       for k in range(NBUF)]
                for k in range(NBUF):
                    cps[k].wait()
                    pltpu.sync_copy(rows[k], acc.at[dst_v.at[g * NBUF + k]],
                                    add=True)

        plsc.subcore_barrier()
        orow = pl.multiple_of(c * NP + s * RPT, 8)
        pltpu.sync_copy(acc.at[pl.ds(s * RPT, RPT)], out_r.at[pl.ds(orow, RPT)])

    k = pl.kernel(body, out_type=(out_type,), mesh=mesh,
                  scratch_types=scratch)
    zw = jnp.zeros((RPT, width), jnp.float32)
    out = k(table, idx_src, idx_dst, zw)[0]
    return out.reshape(NCORE, NP, width)


def _dot(a, b):
    return jnp.dot(a, b, preferred_element_type=jnp.float32)


def _tc1(aggp, x, Wl, bl, Wr):
    def body(aggp_r, x_r, wl_r, bl_r, wr_r, h_r, invc_r):
        cnt16 = aggp_r[1][:, :16]
        invc16 = 1.0 / jnp.maximum(cnt16, 1.0)
        mean = aggp_r[0] * invc16[:, :1]
        out = _dot(mean, wl_r[...]) + bl_r[...][None, :] + _dot(x_r[...], wr_r[...])
        out = jnp.maximum(out, 0.0)
        h_r[0] = out[:, :128]
        h_r[1] = out[:, 128:]
        invc_r[...] = invc16

    return pl.pallas_call(
        body,
        grid=(GRID,),
        in_specs=[
            pl.BlockSpec((2, BN, 128), lambda i: (0, i, 0)),
            pl.BlockSpec((BN, 128), lambda i: (i, 0)),
            pl.BlockSpec((128, 256), lambda i: (0, 0)),
            pl.BlockSpec((256,), lambda i: (0,)),
            pl.BlockSpec((128, 256), lambda i: (0, 0)),
        ],
        out_specs=[
            pl.BlockSpec((2, BN, 128), lambda i: (0, i, 0)),
            pl.BlockSpec((BN, 16), lambda i: (i, 0)),
        ],
        out_shape=[
            jax.ShapeDtypeStruct((2, NP, 128), jnp.float32),
            jax.ShapeDtypeStruct((NP, 16), jnp.float32),
        ],
    )(aggp, x, Wl, bl, Wr)


def _tc_mid(aggp, hp, invc, Wl, bl, Wr):
    def body(aggp_r, hp_r, invc_r, wl_r, bl_r, wr_r, h_r):
        iv = invc_r[:, :1]
        out = (_dot(aggp_r[0] * iv, wl_r[:128])
               + _dot(aggp_r[1] * iv, wl_r[128:])
               + bl_r[...][None, :]
               + _dot(hp_r[0], wr_r[:128])
               + _dot(hp_r[1], wr_r[128:]))
        out = jnp.maximum(out, 0.0)
        h_r[0] = out[:, :128]
        h_r[1] = out[:, 128:]

    return pl.pallas_call(
        body,
        grid=(GRID,),
        in_specs=[
            pl.BlockSpec((2, BN, 128), lambda i: (0, i, 0)),
            pl.BlockSpec((2, BN, 128), lambda i: (0, i, 0)),
            pl.BlockSpec((BN, 16), lambda i: (i, 0)),
            pl.BlockSpec((256, 256), lambda i: (0, 0)),
            pl.BlockSpec((256,), lambda i: (0,)),
            pl.BlockSpec((256, 256), lambda i: (0, 0)),
        ],
        out_specs=[pl.BlockSpec((2, BN, 128), lambda i: (0, i, 0))],
        out_shape=[jax.ShapeDtypeStruct((2, NP, 128), jnp.float32)],
    )(aggp, hp, invc, Wl, bl, Wr)[0]


def _tc3(aggp, hp, invc, Wl, bl, Wr, Wl4, bl4, Wr4):
    def body(aggp_r, hp_r, invc_r, wl_r, bl_r, wr_r, wl4_r, bl4_r, wr4_r,
             y_r, r_r):
        iv = invc_r[:, :1]
        h3 = (_dot(aggp_r[0] * iv, wl_r[:128])
              + _dot(aggp_r[1] * iv, wl_r[128:])
              + bl_r[...][None, :]
              + _dot(hp_r[0], wr_r[:128])
              + _dot(hp_r[1], wr_r[128:]))
        h3 = jnp.maximum(h3, 0.0)
        y_r[...] = _dot(h3, wl4_r[...])
        r_r[...] = _dot(h3, wr4_r[...]) + bl4_r[...][None, :]

    return pl.pallas_call(
        body,
        grid=(GRID,),
        in_specs=[
            pl.BlockSpec((2, BN, 128), lambda i: (0, i, 0)),
            pl.BlockSpec((2, BN, 128), lambda i: (0, i, 0)),
            pl.BlockSpec((BN, 16), lambda i: (i, 0)),
            pl.BlockSpec((256, 256), lambda i: (0, 0)),
            pl.BlockSpec((256,), lambda i: (0,)),
            pl.BlockSpec((256, 256), lambda i: (0, 0)),
            pl.BlockSpec((256, 128), lambda i: (0, 0)),
            pl.BlockSpec((128,), lambda i: (0,)),
            pl.BlockSpec((256, 128), lambda i: (0, 0)),
        ],
        out_specs=[
            pl.BlockSpec((BN, 128), lambda i: (i, 0)),
            pl.BlockSpec((BN, 128), lambda i: (i, 0)),
        ],
        out_shape=[
            jax.ShapeDtypeStruct((NP, 128), jnp.float32),
            jax.ShapeDtypeStruct((NP, 128), jnp.float32),
        ],
    )(aggp, hp, invc, Wl, bl, Wr, Wl4, bl4, Wr4)


def _tc4(aggp, r, invc):
    def body(aggp_r, r_r, invc_r, out_r):
        out_r[...] = (aggp_r[0] + aggp_r[1]) * invc_r[:, :1] + r_r[...]

    return pl.pallas_call(
        body,
        grid=(GRID,),
        in_specs=[
            pl.BlockSpec((2, BN, 128), lambda i: (0, i, 0)),
            pl.BlockSpec((BN, 128), lambda i: (i, 0)),
            pl.BlockSpec((BN, 16), lambda i: (i, 0)),
        ],
        out_specs=[pl.BlockSpec((BN, 128), lambda i: (i, 0))],
        out_shape=[jax.ShapeDtypeStruct((NP, 128), jnp.float32)],
    )(aggp, r, invc)[0]


def kernel(x, edge_index, Wl1, bl1, Wr1, Wl2, bl2, Wr2, Wl3, bl3, Wr3,
           Wl4, bl4, Wr4):
    src = edge_index[0]
    dst = edge_index[1]
    src2d = jnp.full((EP,), N, jnp.int32).at[:E].set(src).reshape(NCHUNK, CH)
    dst2d = jnp.full((EP,), N, jnp.int32).at[:E].set(dst).reshape(NCHUNK, CH)
    src_fs = jnp.concatenate([src2d, src2d + NP], axis=0)
    xp = jnp.zeros((NP, 128), jnp.float32).at[:N].set(x)
    x1 = jnp.concatenate([xp, jnp.ones((1, 128), jnp.float32),
                          jnp.zeros((NP - 1, 128), jnp.float32)], axis=0)
    src_fs1 = jnp.concatenate([src2d, jnp.full((NCHUNK, CH), NP, jnp.int32)],
                              axis=0)

    aggp = _seg_sum(x1, src_fs1, dst2d, 128, es=False)
    h1, invc = _tc1(aggp, xp, Wl1, bl1, Wr1)
    agg2 = _seg_sum(h1.reshape(NCORE * NP, 128), src_fs, dst2d, 128, es=False)
    h2 = _tc_mid(agg2, h1, invc, Wl2, bl2, Wr2)
    agg3 = _seg_sum(h2.reshape(NCORE * NP, 128), src_fs, dst2d, 128, es=False)
    y, r = _tc3(agg3, h2, invc, Wl3, bl3, Wr3, Wl4, bl4, Wr4)
    agg4 = _seg_sum(y, src2d, dst2d, 128, es=True)
    out = _tc4(agg4, r, invc)
    return out[:N]


# speedup vs baseline: 5.2512x; 5.2512x over previous
import functools

import jax
import jax.numpy as jnp
from jax import lax
from jax.experimental import pallas as pl
from jax.experimental.pallas import tpu as pltpu
from jax.experimental.pallas import tpu_sc as plsc

N = 10000
NP = 10112
E = 320000
CH = 128
NCHUNK = 2560
EP = NCHUNK * CH
KB = 16
NBLK = NCHUNK // KB
NSUB = 16
NCORE = 2
RPT = NP // NSUB
BN = 632
GRID = NP // BN


def _seg_sum(table, idx_src, idx_dst, width, es):
    nblk = NBLK // (NCORE * NSUB) if es else NBLK // NSUB
    out_type = jax.ShapeDtypeStruct((NCORE * NP, width), jnp.float32)
    NBUF = 2
    scratch = [
        pltpu.VMEM((KB, CH), jnp.int32),
        pltpu.VMEM((KB, CH), jnp.int32),
    ] + [pltpu.VMEM((CH, width), jnp.float32) for _ in range(NBUF)] + [
        pltpu.VMEM_SHARED((NP, width), jnp.float32),
    ] + [pltpu.SemaphoreType.DMA for _ in range(NBUF)]

    mesh = plsc.VectorSubcoreMesh(core_axis_name="c", subcore_axis_name="s",
                                  num_cores=NCORE, num_subcores=NSUB)

    def body(table_r, src_r, dst_r, zw_r, out_r, src_v, dst_v, *rest):
        rows = rest[:NBUF]
        acc = rest[NBUF]
        sems = rest[NBUF + 1:]
        c = lax.axis_index("c")
        s = lax.axis_index("s")
        pltpu.sync_copy(zw_r, acc.at[pl.ds(s * RPT, RPT)])
        if es:
            src_base = (s * NCORE + c) * nblk * KB
            dst_base = src_base
        else:
            src_base = c * NCHUNK + s * nblk * KB
            dst_base = s * nblk * KB
        plsc.subcore_barrier()

        @pl.loop(0, nblk)
        def _blk(b):
            srow = pl.multiple_of(src_base + b * KB, 8)
            drow = pl.multiple_of(dst_base + b * KB, 8)
            pltpu.sync_copy(src_r.at[pl.ds(srow, KB)], src_v)
            pltpu.sync_copy(dst_r.at[pl.ds(drow, KB)], dst_v)

            @pl.loop(0, KB // NBUF)
            def _grp(g):
                cps = [pltpu.async_copy(table_r.at[src_v.at[g * NBUF + k]],
                                        rows[k], sems[k])
                       for k in range(NBUF)]
                for k in range(NBUF):
                    cps[k].wait()
                    pltpu.sync_copy(rows[k], acc.at[dst_v.at[g * NBUF + k]],
                                    add=True)

        plsc.subcore_barrier()
        orow = pl.multiple_of(c * NP + s * RPT, 8)
        pltpu.sync_copy(acc.at[pl.ds(s * RPT, RPT)], out_r.at[pl.ds(orow, RPT)])

    k = pl.kernel(body, out_type=(out_type,), mesh=mesh,
                  scratch_types=scratch)
    zw = jnp.zeros((RPT, width), jnp.float32)
    out = k(table, idx_src, idx_dst, zw)[0]
    return out.reshape(NCORE, NP, width)


def _dot(a, b):
    return jnp.dot(a, b, preferred_element_type=jnp.float32)


def _tc1(aggp, x, Wl, bl, Wr):
    def body(aggp_r, x_r, wl_r, bl_r, wr_r, h_r, invc_r):
        cnt16 = aggp_r[1][:, :16]
        invc16 = 1.0 / jnp.maximum(cnt16, 1.0)
        mean = aggp_r[0] * invc16[:, :1]
        out = _dot(mean, wl_r[...]) + bl_r[...][None, :] + _dot(x_r[...], wr_r[...])
        out = jnp.maximum(out, 0.0)
        h_r[0] = out[:, :128]
        h_r[1] = out[:, 128:]
        invc_r[...] = invc16

    return pl.pallas_call(
        body,
        grid=(GRID,),
        in_specs=[
            pl.BlockSpec((2, BN, 128), lambda i: (0, i, 0)),
            pl.BlockSpec((BN, 128), lambda i: (i, 0)),
            pl.BlockSpec((128, 256), lambda i: (0, 0)),
            pl.BlockSpec((256,), lambda i: (0,)),
            pl.BlockSpec((128, 256), lambda i: (0, 0)),
        ],
        out_specs=[
            pl.BlockSpec((2, BN, 128), lambda i: (0, i, 0)),
            pl.BlockSpec((BN, 16), lambda i: (i, 0)),
        ],
        out_shape=[
            jax.ShapeDtypeStruct((2, NP, 128), jnp.float32),
            jax.ShapeDtypeStruct((NP, 16), jnp.float32),
        ],
    )(aggp, x, Wl, bl, Wr)


def _tc_mid(aggp, hp, invc, Wl, bl, Wr):
    def body(aggp_r, hp_r, invc_r, wl_r, bl_r, wr_r, h_r):
        iv = invc_r[:, :1]
        out = (_dot(aggp_r[0] * iv, wl_r[:128])
               + _dot(aggp_r[1] * iv, wl_r[128:])
               + bl_r[...][None, :]
               + _dot(hp_r[0], wr_r[:128])
               + _dot(hp_r[1], wr_r[128:]))
        out = jnp.maximum(out, 0.0)
        h_r[0] = out[:, :128]
        h_r[1] = out[:, 128:]

    return pl.pallas_call(
        body,
        grid=(GRID,),
        in_specs=[
            pl.BlockSpec((2, BN, 128), lambda i: (0, i, 0)),
            pl.BlockSpec((2, BN, 128), lambda i: (0, i, 0)),
            pl.BlockSpec((BN, 16), lambda i: (i, 0)),
            pl.BlockSpec((256, 256), lambda i: (0, 0)),
            pl.BlockSpec((256,), lambda i: (0,)),
            pl.BlockSpec((256, 256), lambda i: (0, 0)),
        ],
        out_specs=[pl.BlockSpec((2, BN, 128), lambda i: (0, i, 0))],
        out_shape=[jax.ShapeDtypeStruct((2, NP, 128), jnp.float32)],
    )(aggp, hp, invc, Wl, bl, Wr)[0]


def _tc3(aggp, hp, invc, Wl, bl, Wr, Wl4, bl4, Wr4):
    def body(aggp_r, hp_r, invc_r, wl_r, bl_r, wr_r, wl4_r, bl4_r, wr4_r,
             y_r, r_r):
        iv = invc_r[:, :1]
        h3 = (_dot(aggp_r[0] * iv, wl_r[:128])
              + _dot(aggp_r[1] * iv, wl_r[128:])
              + bl_r[...][None, :]
              + _dot(hp_r[0], wr_r[:128])
              + _dot(hp_r[1], wr_r[128:]))
        h3 = jnp.maximum(h3, 0.0)
        y_r[...] = _dot(h3, wl4_r[...])
        r_r[...] = _dot(h3, wr4_r[...]) + bl4_r[...][None, :]

    return pl.pallas_call(
        body,
        grid=(GRID,),
        in_specs=[
            pl.BlockSpec((2, BN, 128), lambda i: (0, i, 0)),
            pl.BlockSpec((2, BN, 128), lambda i: (0, i, 0)),
            pl.BlockSpec((BN, 16), lambda i: (i, 0)),
            pl.BlockSpec((256, 256), lambda i: (0, 0)),
            pl.BlockSpec((256,), lambda i: (0,)),
            pl.BlockSpec((256, 256), lambda i: (0, 0)),
            pl.BlockSpec((256, 128), lambda i: (0, 0)),
            pl.BlockSpec((128,), lambda i: (0,)),
            pl.BlockSpec((256, 128), lambda i: (0, 0)),
        ],
        out_specs=[
            pl.BlockSpec((BN, 128), lambda i: (i, 0)),
            pl.BlockSpec((BN, 128), lambda i: (i, 0)),
        ],
        out_shape=[
            jax.ShapeDtypeStruct((NP, 128), jnp.float32),
            jax.ShapeDtypeStruct((NP, 128), jnp.float32),
        ],
    )(aggp, hp, invc, Wl, bl, Wr, Wl4, bl4, Wr4)


def _tc4(aggp, r, invc):
    def body(aggp_r, r_r, invc_r, out_r):
        out_r[...] = (aggp_r[0] + aggp_r[1]) * invc_r[:, :1] + r_r[...]

    return pl.pallas_call(
        body,
        grid=(GRID,),
        in_specs=[
            pl.BlockSpec((2, BN, 128), lambda i: (0, i, 0)),
            pl.BlockSpec((BN, 128), lambda i: (i, 0)),
            pl.BlockSpec((BN, 16), lambda i: (i, 0)),
        ],
        out_specs=[pl.BlockSpec((BN, 128), lambda i: (i, 0))],
        out_shape=[jax.ShapeDtypeStruct((NP, 128), jnp.float32)],
    )(aggp, r, invc)[0]


def kernel(x, edge_index, Wl1, bl1, Wr1, Wl2, bl2, Wr2, Wl3, bl3, Wr3,
           Wl4, bl4, Wr4):
    src = edge_index[0]
    dst = edge_index[1]
    src2d = jnp.full((EP,), N, jnp.int32).at[:E].set(src).reshape(NCHUNK, CH)
    dst2d = jnp.full((EP,), N, jnp.int32).at[:E].set(dst).reshape(NCHUNK, CH)
    src_fs = jnp.concatenate([src2d, src2d + NP], axis=0)
    xp = jnp.zeros((NP, 128), jnp.float32).at[:N].set(x)
    x1 = jnp.concatenate([xp, jnp.ones((NP, 128), jnp.float32)], axis=0)

    aggp = _seg_sum(x1, src_fs, dst2d, 128, es=False)
    h1, invc = _tc1(aggp, xp, Wl1, bl1, Wr1)
    agg2 = _seg_sum(h1.reshape(NCORE * NP, 128), src_fs, dst2d, 128, es=False)
    h2 = _tc_mid(agg2, h1, invc, Wl2, bl2, Wr2)
    agg3 = _seg_sum(h2.reshape(NCORE * NP, 128), src_fs, dst2d, 128, es=False)
    y, r = _tc3(agg3, h2, invc, Wl3, bl3, Wr3, Wl4, bl4, Wr4)
    agg4 = _seg_sum(y, src2d, dst2d, 128, es=True)
    out = _tc4(agg4, r, invc)
    return out[:N]
